# baseline (speedup 1.0000x reference)
import functools

import jax
import jax.numpy as jnp
from jax import lax
from jax.experimental import pallas as pl
from jax.experimental.pallas import tpu as pltpu
from jax.experimental.pallas import tpu_sc as plsc

B, F, T = 4096, 128, 100
N = B * F
NC, NS, L = 2, 16, 16
NW = NC * NS
ROWS_W = N // NW
R = 256
NCHUNK = ROWS_W // R
CW = R * T


def _spike_times(xv):
    xn = jnp.minimum(jnp.maximum((xv + 1.0) * 0.5, 0.0), 1.0)
    v = (1.0 - xn) * 99.0
    fv = v + 0.5
    ti = fv.astype(jnp.int32)
    tie = ti.astype(jnp.float32) == fv
    odd = (ti & 1) == 1
    ti = ti - jnp.where(tie & odd, 1, 0)
    return jnp.minimum(jnp.maximum(ti, 0), T - 1)


@functools.partial(
    pl.kernel,
    out_type=jax.ShapeDtypeStruct((T, N), jnp.float32),
    mesh=plsc.VectorSubcoreMesh(core_axis_name="c", subcore_axis_name="s"),
    compiler_params=pltpu.CompilerParams(needs_layout_passes=False),
    scratch_types=[
        pltpu.VMEM((ROWS_W,), jnp.float32),
        pltpu.VMEM((T, R), jnp.float32),
        pltpu.VMEM((T, R), jnp.float32),
        pltpu.VMEM((R,), jnp.int32),
        pltpu.VMEM((R,), jnp.int32),
        pltpu.SemaphoreType.DMA,
        pltpu.SemaphoreType.DMA,
    ],
)
def _encode(x_hbm, out_hbm, xbuf, ob0, ob1, ib0, ib1, sem0, sem1):
    wid = lax.axis_index("s") * NC + lax.axis_index("c")
    row0 = wid * ROWS_W
    pltpu.sync_copy(x_hbm.at[pl.ds(row0, ROWS_W)], xbuf)

    zeros = jnp.zeros((L,), jnp.float32)
    ones = jnp.full((L,), 1.0, jnp.float32)
    lanes = lax.iota(jnp.int32, L)

    def _zero_init(i, _):
        q = i * L + lanes
        plsc.store_scatter(ob0, [q // R, q % R], zeros)
        plsc.store_scatter(ob1, [q // R, q % R], zeros)
        return 0

    lax.fori_loop(0, CW // L, _zero_init, 0)

    obufs, ibufs, sems = (ob0, ob1), (ib0, ib1), (sem0, sem1)
    copies = [None] * NCHUNK
    for c in range(NCHUNK):
        p = c & 1
        ob, ib = obufs[p], ibufs[p]
        if c >= 2:
            copies[c - 2].wait()

            def _rezero(j, _, ob=ob, ib=ib):
                idx_t = ib[pl.ds(j * L, L)]
                plsc.store_scatter(ob, [idx_t, j * L + lanes], zeros)
                return 0

            lax.fori_loop(0, R // L, _rezero, 0)

        def _set_ones(j, _, ob=ob, ib=ib, c=c):
            xv = xbuf[pl.ds(c * R + j * L, L)]
            ti = _spike_times(xv)
            plsc.store_scatter(ob, [ti, j * L + lanes], ones)
            ib[pl.ds(j * L, L)] = ti
            return 0

        lax.fori_loop(0, R // L, _set_ones, 0)
        dst = out_hbm.at[:, pl.ds(row0 + c * R, R)]
        copies[c] = pltpu.async_copy(ob, dst, sems[p])

    copies[NCHUNK - 2].wait()
    copies[NCHUNK - 1].wait()


def kernel(x):
    o = _encode(x.reshape(N))
    return jnp.transpose(o.reshape(T, B, F), (1, 2, 0))


# speedup vs baseline: 2.7570x; 2.7570x over previous
import functools

import jax
import jax.numpy as jnp
from jax import lax
from jax.experimental import pallas as pl
from jax.experimental.pallas import tpu as pltpu
from jax.experimental.pallas import tpu_sc as plsc

B, F, T = 4096, 128, 100
N = B * F
NC, NS, L = 2, 16, 16
NW = NC * NS
ROWS_W = N // NW
PB = 8
GR = PB * F
NG = ROWS_W // GR
TSUB = 25
NT = T // TSUB
NCHUNK = NG * NT
CW = TSUB * GR
JG = GR // L


def _spike_times(xv):
    xn = jnp.minimum(jnp.maximum((xv + 1.0) * 0.5, 0.0), 1.0)
    v = (1.0 - xn) * 99.0
    fv = v + 0.5
    ti = fv.astype(jnp.int32)
    tie = ti.astype(jnp.float32) == fv
    odd = (ti & 1) == 1
    ti = ti - jnp.where(tie & odd, 1, 0)
    return jnp.minimum(jnp.maximum(ti, 0), T - 1)


@functools.partial(
    pl.kernel,
    out_type=jax.ShapeDtypeStruct((T, B, F), jnp.float32),
    mesh=plsc.VectorSubcoreMesh(core_axis_name="c", subcore_axis_name="s"),
    compiler_params=pltpu.CompilerParams(needs_layout_passes=False),
    scratch_types=[
        pltpu.VMEM((ROWS_W,), jnp.float32),
        pltpu.VMEM((TSUB, PB, F), jnp.float32),
        pltpu.VMEM((TSUB, PB, F), jnp.float32),
        pltpu.VMEM((GR,), jnp.int32),
        pltpu.VMEM((GR,), jnp.int32),
        pltpu.SemaphoreType.DMA,
        pltpu.SemaphoreType.DMA,
    ],
)
def _encode(x_hbm, out_hbm, xbuf, ob0, ob1, tb0, tb1, sem0, sem1):
    wid = lax.axis_index("s") * NC + lax.axis_index("c")
    row0 = wid * ROWS_W
    plane0 = wid * (B // NW)
    pltpu.sync_copy(x_hbm.at[pl.ds(row0, ROWS_W)], xbuf)

    zeros = jnp.zeros((L,), jnp.float32)
    ones = jnp.full((L,), 1.0, jnp.float32)
    lanes = lax.iota(jnp.int32, L)

    def _zero_init(i, _):
        q = i * L + lanes
        qt, qr = q // (PB * F), q % (PB * F)
        plsc.store_scatter(ob0, [qt, qr // F, qr % F], zeros)
        plsc.store_scatter(ob1, [qt, qr // F, qr % F], zeros)
        return 0

    lax.fori_loop(0, CW // L, _zero_init, 0)

    def _scatter_pass(ob, tb, t0, val):
        def body(j, _):
            ti = tb[pl.ds(j * L, L)]
            m = (ti >= t0) & (ti < t0 + TSUB)
            dt = jnp.minimum(jnp.maximum(ti - t0, 0), TSUB - 1)
            idx_b = jnp.full((L,), 0, jnp.int32) + j // PB
            idx_f = (j % PB) * L + lanes
            plsc.store_scatter(ob, [dt, idx_b, idx_f], val, mask=m)
            return 0

        lax.fori_loop(0, JG, body, 0)

    obufs, tbufs, sems = (ob0, ob1), (tb0, tb1), (sem0, sem1)
    copies = [None] * NCHUNK
    for k in range(NCHUNK):
        g, c = k // NT, k % NT
        p = k & 1
        ob, tb = obufs[p], tbufs[g & 1]
        if c == 0:
            def _times(j, _, tb=tb, g=g):
                xv = xbuf[pl.ds(g * GR + j * L, L)]
                tb[pl.ds(j * L, L)] = _spike_times(xv)
                return 0

            lax.fori_loop(0, JG, _times, 0)
        if k >= 2:
            copies[k - 2].wait()
            g2, c2 = (k - 2) // NT, (k - 2) % NT
            _scatter_pass(ob, tbufs[g2 & 1], c2 * TSUB, zeros)
        _scatter_pass(ob, tb, c * TSUB, ones)
        dst = out_hbm.at[pl.ds(c * TSUB, TSUB), pl.ds(plane0 + g * PB, PB), :]
        copies[k] = pltpu.async_copy(ob, dst, sems[p])

    copies[NCHUNK - 2].wait()
    copies[NCHUNK - 1].wait()


def kernel(x):
    return jnp.transpose(_encode(x.reshape(N)), (1, 2, 0))
